# Initial kernel scaffold; baseline (speedup 1.0000x reference)
import jax
import jax.numpy as jnp
from jax.experimental import pallas as pl


def kernel(x, edge_index, edge_attr, batch, node_emb, edge_emb0, edge_emb1, w1_0, b1_0, w2_0, b2_0, w1_1, b1_1, w2_1, b2_1, lstm_wih, lstm_whh, lstm_bih, lstm_bhh, wp1, bp1, wp2, bp2):
    raise NotImplementedError("write your pallas kernel here")


# speedup vs baseline: 5.1261x; 5.1261x over previous
import functools

import jax
import jax.numpy as jnp
from jax import lax
from jax.experimental import pallas as pl
from jax.experimental.pallas import tpu as pltpu
from jax.experimental.pallas import tpu_sc as plsc

N = 10000
E = 320000
D = 128
B = 64
NA = 5
NC = 2
NS = 16
NW = NC * NS
EPT = E // NS
CHUNK = 400
NCHUNK = EPT // CHUNK
HALF = 5120
AGG_ROWS = HALF + 8
ROWS_PER_TILE = HALF // NS


def _sc_edge_body(mt_hbm, cmb_hbm, dst_hbm, out_hbm,
                  dst_v, cmb_v, rows_v, agg_sh, sem):
    c = lax.axis_index("c")
    s = lax.axis_index("s")
    lo = c * HALF

    zero = jnp.zeros((16,), jnp.float32)

    def zloop(i, _):
        rows_v[i // 8, pl.ds((i % 8) * 16, 16)] = zero
        return 0

    lax.fori_loop(0, CHUNK * (D // 16), zloop, 0, unroll=8)

    stripe = s * ROWS_PER_TILE
    pltpu.sync_copy(rows_v.at[pl.ds(0, ROWS_PER_TILE)],
                    agg_sh.at[pl.ds(stripe, ROWS_PER_TILE)])
    @pl.when(s == 0)
    def _():
        pltpu.sync_copy(rows_v.at[pl.ds(0, 8)], agg_sh.at[pl.ds(HALF, 8)])

    plsc.subcore_barrier()

    def chunk_body(k, _):
        base = s * EPT + k * CHUNK
        pltpu.sync_copy(cmb_hbm.at[pl.ds(base, CHUNK)], cmb_v)
        pltpu.sync_copy(dst_hbm.at[pl.ds(base, CHUNK)], dst_v)
        for j in range(CHUNK // 16):
            sl = pl.ds(j * 16, 16)
            d = dst_v[sl] - lo
            in_half = (d >= 0) & (d < HALF)
            dst_v[sl] = jnp.where(in_half, d, HALF)
        pltpu.async_copy(mt_hbm.at[cmb_v], rows_v, sem).wait()
        pltpu.sync_copy(rows_v, agg_sh.at[dst_v], add=True)
        return 0

    lax.fori_loop(0, NCHUNK, chunk_body, 0)
    plsc.subcore_barrier()

    pltpu.sync_copy(agg_sh.at[pl.ds(stripe, ROWS_PER_TILE)],
                    rows_v.at[pl.ds(0, ROWS_PER_TILE)])
    pltpu.sync_copy(rows_v.at[pl.ds(0, ROWS_PER_TILE)],
                    out_hbm.at[c, pl.ds(stripe, ROWS_PER_TILE)])


@jax.jit
def _sc_edge_agg(msg_table, cmb, dst):
    mesh = plsc.VectorSubcoreMesh(core_axis_name="c", subcore_axis_name="s")
    return pl.kernel(
        _sc_edge_body,
        out_type=jax.ShapeDtypeStruct((NC, HALF, D), jnp.float32),
        mesh=mesh,
        scratch_types=[
            pltpu.VMEM((CHUNK,), jnp.int32),
            pltpu.VMEM((CHUNK,), jnp.int32),
            pltpu.VMEM((CHUNK, D), jnp.float32),
            pltpu.VMEM_SHARED((AGG_ROWS, D), jnp.float32),
            pltpu.SemaphoreType.DMA,
        ],
    )(msg_table, cmb, dst)


NB = 2000
NGRID = N // NB


def _tc_embed_body(x_ref, emb_ref, ee_ref, h_ref, mt_ref):
    xb = x_ref[0, 0, :]
    onehot = (xb[:, None] == lax.broadcasted_iota(jnp.int32, (NB, 128), 1))
    h = jnp.dot(onehot.astype(jnp.float32), emb_ref[...],
                preferred_element_type=jnp.float32)
    h_ref[...] = h
    mt_ref[...] = jax.nn.relu(h[:, None, :] + ee_ref[...][None, :, :])


@jax.jit
def _tc_embed(x, node_emb_pad, ee):
    return pl.pallas_call(
        _tc_embed_body,
        grid=(NGRID,),
        in_specs=[
            pl.BlockSpec((1, 1, NB), lambda i: (i, 0, 0)),
            pl.BlockSpec((128, D), lambda i: (0, 0)),
            pl.BlockSpec((NA, D), lambda i: (0, 0)),
        ],
        out_specs=[
            pl.BlockSpec((NB, D), lambda i: (i, 0)),
            pl.BlockSpec((NB, NA, D), lambda i: (i, 0, 0)),
        ],
        out_shape=[
            jax.ShapeDtypeStruct((N, D), jnp.float32),
            jax.ShapeDtypeStruct((N, NA, D), jnp.float32),
        ],
    )(x, node_emb_pad, ee)


def _tc_mlp_body(h_ref, agg_ref, w1_ref, b1_ref, w2_ref, b2_ref, ee_ref,
                 hn_ref, mt_ref, *, relu_out, emit_table):
    z = h_ref[...] + agg_ref[...]
    z = jnp.dot(z, w1_ref[...], preferred_element_type=jnp.float32) + b1_ref[...]
    z = jax.nn.relu(z)
    z = jnp.dot(z, w2_ref[...], preferred_element_type=jnp.float32) + b2_ref[...]
    if relu_out:
        z = jax.nn.relu(z)
    hn_ref[...] = z
    if emit_table:
        mt_ref[...] = jax.nn.relu(z[:, None, :] + ee_ref[...][None, :, :])


@functools.partial(jax.jit, static_argnames=("relu_out", "emit_table"))
def _tc_mlp(h, agg, w1, b1, w2, b2, ee, relu_out, emit_table):
    return pl.pallas_call(
        functools.partial(_tc_mlp_body, relu_out=relu_out, emit_table=emit_table),
        grid=(NGRID,),
        in_specs=[
            pl.BlockSpec((NB, D), lambda i: (i, 0)),
            pl.BlockSpec((NB, D), lambda i: (i, 0)),
            pl.BlockSpec((D, 2 * D), lambda i: (0, 0)),
            pl.BlockSpec((2 * D,), lambda i: (0,)),
            pl.BlockSpec((2 * D, D), lambda i: (0, 0)),
            pl.BlockSpec((D,), lambda i: (0,)),
            pl.BlockSpec((NA, D), lambda i: (0, 0)),
        ],
        out_specs=[
            pl.BlockSpec((NB, D), lambda i: (i, 0)),
            pl.BlockSpec((NB, NA, D), lambda i: (i, 0, 0)),
        ],
        out_shape=[
            jax.ShapeDtypeStruct((N, D), jnp.float32),
            jax.ShapeDtypeStruct((N, NA, D), jnp.float32),
        ],
    )(h, agg, w1, b1, w2, b2, ee)


def _tc_s2s_body(h_ref, batch_ref, wih_ref, whh_ref, bih_ref, bhh_ref,
                 wp1_ref, bp1_ref, wp2_ref, bp2_ref, out_ref):
    h = h_ref[...]
    batch = batch_ref[...]
    S = (batch[:, None] == lax.broadcasted_iota(jnp.int32, (N, 128), 1))
    Sf = S.astype(jnp.float32)

    NEG = jnp.float32(-1e30)
    qstar = jnp.zeros((128, 2 * D), jnp.float32)
    hh = jnp.zeros((128, D), jnp.float32)
    cc = jnp.zeros((128, D), jnp.float32)
    for _ in range(2):
        gates = (jnp.dot(qstar, wih_ref[...], preferred_element_type=jnp.float32)
                 + bih_ref[...]
                 + jnp.dot(hh, whh_ref[...], preferred_element_type=jnp.float32)
                 + bhh_ref[...])
        ig = jax.nn.sigmoid(gates[:, 0 * D:1 * D])
        fg = jax.nn.sigmoid(gates[:, 1 * D:2 * D])
        gg = jnp.tanh(gates[:, 2 * D:3 * D])
        og = jax.nn.sigmoid(gates[:, 3 * D:4 * D])
        cc = fg * cc + ig * gg
        hh = og * jnp.tanh(cc)
        q = hh
        qb = jnp.dot(Sf, q, preferred_element_type=jnp.float32)
        e = jnp.sum(h * qb, axis=1)
        em = jnp.where(S, e[:, None], NEG)
        m = jnp.max(em, axis=0)
        mb = jnp.dot(Sf, m[:, None], preferred_element_type=jnp.float32)[:, 0]
        ex = jnp.exp(e - mb)
        ssum = jnp.sum(Sf * ex[:, None], axis=0)
        sb = jnp.dot(Sf, ssum[:, None], preferred_element_type=jnp.float32)[:, 0]
        a = ex / (sb + 1e-16)
        r = lax.dot_general(Sf, a[:, None] * h,
                            (((0,), (0,)), ((), ())),
                            preferred_element_type=jnp.float32)
        qstar = jnp.concatenate([q, r], axis=1)

    out = jnp.dot(qstar, wp1_ref[...], preferred_element_type=jnp.float32) + bp1_ref[...]
    out = jax.nn.relu(out)
    out = jnp.dot(out, wp2_ref[...], preferred_element_type=jnp.float32) + bp2_ref[...]
    o0 = out[:, 0:1]
    o1 = out[:, 1:2]
    mx = jnp.maximum(o0, o1)
    e0 = jnp.exp(o0 - mx)
    e1 = jnp.exp(o1 - mx)
    tot = e0 + e1
    probs = jnp.concatenate([e0 / tot, e1 / tot], axis=1)
    out_ref[...] = jnp.pad(probs, ((0, 0), (0, 126)))


@jax.jit
def _tc_s2s(h, batch, wih, whh, bih, bhh, wp1, bp1, wp2_pad, bp2_pad):
    return pl.pallas_call(
        _tc_s2s_body,
        out_shape=jax.ShapeDtypeStruct((128, 128), jnp.float32),
    )(h, batch, wih, whh, bih, bhh, wp1, bp1, wp2_pad, bp2_pad)


def kernel(x, edge_index, edge_attr, batch, node_emb, edge_emb0, edge_emb1,
           w1_0, b1_0, w2_0, b2_0, w1_1, b1_1, w2_1, b2_1,
           lstm_wih, lstm_whh, lstm_bih, lstm_bhh, wp1, bp1, wp2, bp2):
    x = x.astype(jnp.int32)
    src = edge_index[0].astype(jnp.int32)
    dst = edge_index[1].astype(jnp.int32)
    attr = edge_attr.astype(jnp.int32)
    batch = batch.astype(jnp.int32)

    node_emb_pad = jnp.pad(node_emb, ((0, 128 - node_emb.shape[0]), (0, 0)))
    wp2_pad = jnp.pad(wp2, ((0, 0), (0, 126)))
    bp2_pad = jnp.pad(bp2, ((0, 126),))

    h0, mt0 = _tc_embed(x.reshape(NGRID, 1, NB), node_emb_pad, edge_emb0)
    cmb = src * NA + attr
    agg0 = _sc_edge_agg(mt0.reshape(N * NA, D), cmb, dst).reshape(NC * HALF, D)
    h1, mt1 = _tc_mlp(h0, agg0, w1_0, b1_0, w2_0, b2_0, edge_emb1,
                      relu_out=True, emit_table=True)
    agg1 = _sc_edge_agg(mt1.reshape(N * NA, D), cmb, dst).reshape(NC * HALF, D)
    h2, _ = _tc_mlp(h1, agg1, w1_1, b1_1, w2_1, b2_1, edge_emb1,
                    relu_out=False, emit_table=False)
    out = _tc_s2s(h2, batch, lstm_wih, lstm_whh, lstm_bih, lstm_bhh,
                  wp1, bp1, wp2_pad, bp2_pad)
    return out[:B, :2]
